# Initial kernel scaffold; baseline (speedup 1.0000x reference)
import functools

import jax
import jax.numpy as jnp
from jax import lax
from jax.experimental import pallas as pl
from jax.experimental.pallas import tpu as pltpu
from jax.experimental.pallas import tpu_sc as plsc

N = 10000
C = 128
NC, NS = 2, 16
NW = NC * NS
CHUNK = 128
N_PAD = 10016
RPW = N_PAD // NS
DEG_W = 16
ROW_TILE = 400

_mesh = plsc.VectorSubcoreMesh(
    core_axis_name="c", subcore_axis_name="s", num_cores=NC, num_subcores=NS
)


def _worker_id():
    return lax.axis_index("s") * NC + lax.axis_index("c")


def _deg_kernel(epw):
    @functools.partial(
        pl.kernel,
        out_type=jax.ShapeDtypeStruct((NC, N_PAD, DEG_W), jnp.float32),
        mesh=_mesh,
        scratch_types=[
            pltpu.VMEM((CHUNK,), jnp.int32),
            pltpu.VMEM((CHUNK, DEG_W), jnp.float32),
            pltpu.VMEM_SHARED((N_PAD, DEG_W), jnp.float32),
        ],
    )
    def k(dst_hbm, zeros_hbm, out_hbm, idx_v, ones_v, acc_sh):
        cid = lax.axis_index("c")
        sid = lax.axis_index("s")
        wid = _worker_id()

        pltpu.sync_copy(
            zeros_hbm.at[pl.ds(sid * RPW, RPW)], acc_sh.at[pl.ds(sid * RPW, RPW)]
        )

        @pl.loop(0, CHUNK)
        def _(r):
            ones_v[r] = jnp.full((DEG_W,), 1.0, jnp.float32)

        plsc.subcore_barrier()

        @pl.loop(0, epw, step=CHUNK)
        def _(e0):
            base = pl.multiple_of(wid * epw + e0, CHUNK)
            pltpu.sync_copy(dst_hbm.at[pl.ds(base, CHUNK)], idx_v)
            pltpu.sync_copy(ones_v, acc_sh.at[idx_v], add=True)

        plsc.subcore_barrier()
        pltpu.sync_copy(
            acc_sh.at[pl.ds(sid * RPW, RPW)],
            out_hbm.at[cid, pl.ds(sid * RPW, RPW)],
        )

    return k


def _agg_kernel(epw):
    @functools.partial(
        pl.kernel,
        out_type=jax.ShapeDtypeStruct((NC, N_PAD, C), jnp.float32),
        mesh=_mesh,
        scratch_types=[
            pltpu.VMEM((CHUNK,), jnp.int32),
            pltpu.VMEM((CHUNK,), jnp.int32),
            pltpu.VMEM((CHUNK, C), jnp.float32),
            pltpu.VMEM_SHARED((N_PAD, C), jnp.float32),
            pltpu.SemaphoreType.DMA,
        ],
    )
    def k(src_hbm, dst_hbm, table_hbm, zeros_hbm, out_hbm,
          src_v, dst_v, rows_v, acc_sh, sem):
        cid = lax.axis_index("c")
        sid = lax.axis_index("s")
        wid = _worker_id()

        pltpu.sync_copy(
            zeros_hbm.at[pl.ds(sid * RPW, RPW)], acc_sh.at[pl.ds(sid * RPW, RPW)]
        )
        plsc.subcore_barrier()

        @pl.loop(0, epw, step=CHUNK)
        def _(e0):
            base = pl.multiple_of(wid * epw + e0, CHUNK)
            pltpu.sync_copy(src_hbm.at[pl.ds(base, CHUNK)], src_v)
            pltpu.sync_copy(dst_hbm.at[pl.ds(base, CHUNK)], dst_v)
            pltpu.async_copy(table_hbm.at[src_v], rows_v, sem).wait()
            pltpu.sync_copy(rows_v, acc_sh.at[dst_v], add=True)

        plsc.subcore_barrier()
        pltpu.sync_copy(
            acc_sh.at[pl.ds(sid * RPW, RPW)],
            out_hbm.at[cid, pl.ds(sid * RPW, RPW)],
        )

    return k


def _s_from_deg(deg0_ref, deg1_ref):
    deg = deg0_ref[:, :1] + deg1_ref[:, :1] + 1.0
    return lax.rsqrt(deg)


def _mm_scale_body(x_ref, w_ref, deg0_ref, deg1_ref, out_ref):
    s = _s_from_deg(deg0_ref, deg1_ref)
    h = jnp.dot(
        x_ref[...], w_ref[...],
        preferred_element_type=jnp.float32, precision=lax.Precision.HIGHEST,
    )
    out_ref[...] = h * s


def _combine_mm_body(p0_ref, p1_ref, h2_ref, deg0_ref, deg1_ref, b_ref, w_ref,
                     out_ref):
    s = _s_from_deg(deg0_ref, deg1_ref)
    t = s * (p0_ref[...] + p1_ref[...] + h2_ref[...]) + b_ref[...]
    g = jnp.maximum(t, 0.0)
    h = jnp.dot(
        g, w_ref[...],
        preferred_element_type=jnp.float32, precision=lax.Precision.HIGHEST,
    )
    out_ref[...] = h * s


def _final_body(p0_ref, p1_ref, h2_ref, deg0_ref, deg1_ref, b_ref, out_ref):
    s = _s_from_deg(deg0_ref, deg1_ref)
    out_ref[...] = s * (p0_ref[...] + p1_ref[...] + h2_ref[...]) + b_ref[...]


def _row_spec(w):
    return pl.BlockSpec((ROW_TILE, w), lambda i: (i, 0))


def _full_spec(r, w):
    return pl.BlockSpec((r, w), lambda i: (0, 0))


_GRID = (N // ROW_TILE,)
_F32 = jnp.float32


@jax.jit
def kernel(x, edge_index, W1, b1, W2, b2):
    ei = edge_index.astype(jnp.int32)
    src, dst = ei[0], ei[1]
    e = src.shape[0]
    epw = -(-e // (NW * CHUNK)) * CHUNK
    e_pad = epw * NW
    if e_pad != e:
        pad = e_pad - e
        src = jnp.concatenate([src, jnp.zeros((pad,), jnp.int32)])
        dst = jnp.concatenate([dst, jnp.full((pad,), N, jnp.int32)])

    zeros_deg = jnp.zeros((N_PAD, DEG_W), _F32)
    zeros_acc = jnp.zeros((N_PAD, C), _F32)
    b1r = b1.reshape(1, C)
    b2r = b2.reshape(1, C)

    degp = _deg_kernel(epw)(dst, zeros_deg)
    deg0, deg1 = degp[0], degp[1]

    h2_1 = pl.pallas_call(
        _mm_scale_body,
        grid=_GRID,
        in_specs=[
            _row_spec(C), _full_spec(C, C), _row_spec(DEG_W), _row_spec(DEG_W)
        ],
        out_specs=_row_spec(C),
        out_shape=jax.ShapeDtypeStruct((N, C), _F32),
    )(x, W1, deg0, deg1)

    p = _agg_kernel(epw)(src, dst, h2_1, zeros_acc)
    p0, p1 = p[0, :N], p[1, :N]

    h2_2 = pl.pallas_call(
        _combine_mm_body,
        grid=_GRID,
        in_specs=[
            _row_spec(C), _row_spec(C), _row_spec(C),
            _row_spec(DEG_W), _row_spec(DEG_W),
            _full_spec(1, C), _full_spec(C, C),
        ],
        out_specs=_row_spec(C),
        out_shape=jax.ShapeDtypeStruct((N, C), _F32),
    )(p0, p1, h2_1, deg0, deg1, b1r, W2)

    q = _agg_kernel(epw)(src, dst, h2_2, zeros_acc)
    q0, q1 = q[0, :N], q[1, :N]

    out = pl.pallas_call(
        _final_body,
        grid=_GRID,
        in_specs=[
            _row_spec(C), _row_spec(C), _row_spec(C),
            _row_spec(DEG_W), _row_spec(DEG_W),
            _full_spec(1, C),
        ],
        out_specs=_row_spec(C),
        out_shape=jax.ShapeDtypeStruct((N, C), _F32),
    )(q0, q1, h2_2, deg0, deg1, b2r)

    return out


# speedup vs baseline: 10.3548x; 10.3548x over previous
import functools

import jax
import jax.numpy as jnp
from jax import lax
from jax.experimental import pallas as pl
from jax.experimental.pallas import tpu as pltpu
from jax.experimental.pallas import tpu_sc as plsc

N = 10000
C = 128
NC, NS = 2, 16
NW = NC * NS
CHUNK = 128
N_PAD = 10112
RPW = N_PAD // NS
DEG_W = 16
ROW_TILE = 400

_mesh = plsc.VectorSubcoreMesh(
    core_axis_name="c", subcore_axis_name="s", num_cores=NC, num_subcores=NS
)


def _worker_id():
    return lax.axis_index("s") * NC + lax.axis_index("c")


def _deg_kernel(epw):
    @functools.partial(
        pl.kernel,
        out_type=jax.ShapeDtypeStruct((NC, N_PAD, DEG_W), jnp.float32),
        mesh=_mesh,
        scratch_types=[
            pltpu.VMEM((CHUNK,), jnp.int32),
            pltpu.VMEM((CHUNK, DEG_W), jnp.float32),
            pltpu.VMEM_SHARED((N_PAD, DEG_W), jnp.float32),
        ],
    )
    def k(dst_hbm, zeros_hbm, out_hbm, idx_v, ones_v, acc_sh):
        cid = lax.axis_index("c")
        sid = lax.axis_index("s")
        wid = _worker_id()

        pltpu.sync_copy(
            zeros_hbm.at[pl.ds(sid * RPW, RPW)], acc_sh.at[pl.ds(sid * RPW, RPW)]
        )

        @pl.loop(0, CHUNK)
        def _(r):
            ones_v[r] = jnp.full((DEG_W,), 1.0, jnp.float32)

        plsc.subcore_barrier()

        @pl.loop(0, epw, step=CHUNK)
        def _(e0):
            base = pl.multiple_of(wid * epw + e0, CHUNK)
            pltpu.sync_copy(dst_hbm.at[pl.ds(base, CHUNK)], idx_v)
            pltpu.sync_copy(ones_v, acc_sh.at[idx_v], add=True)

        plsc.subcore_barrier()
        pltpu.sync_copy(
            acc_sh.at[pl.ds(sid * RPW, RPW)],
            out_hbm.at[cid, pl.ds(sid * RPW, RPW)],
        )

    return k


def _agg_kernel(epw):
    @functools.partial(
        pl.kernel,
        out_type=jax.ShapeDtypeStruct((NC, N_PAD, C), jnp.float32),
        mesh=_mesh,
        scratch_types=[
            pltpu.VMEM((CHUNK,), jnp.int32),
            pltpu.VMEM((CHUNK,), jnp.int32),
            pltpu.VMEM((CHUNK, C), jnp.float32),
            pltpu.VMEM_SHARED((N_PAD, C), jnp.float32),
            pltpu.SemaphoreType.DMA,
        ],
    )
    def k(src_hbm, dst_hbm, table_hbm, zeros_hbm, out_hbm,
          src_v, dst_v, rows_v, acc_sh, sem):
        cid = lax.axis_index("c")
        sid = lax.axis_index("s")
        wid = _worker_id()

        pltpu.sync_copy(
            zeros_hbm.at[pl.ds(sid * RPW, RPW)], acc_sh.at[pl.ds(sid * RPW, RPW)]
        )
        plsc.subcore_barrier()

        @pl.loop(0, epw, step=CHUNK)
        def _(e0):
            base = pl.multiple_of(wid * epw + e0, CHUNK)
            pltpu.sync_copy(src_hbm.at[pl.ds(base, CHUNK)], src_v)
            pltpu.sync_copy(dst_hbm.at[pl.ds(base, CHUNK)], dst_v)
            pltpu.async_copy(table_hbm.at[src_v], rows_v, sem).wait()
            pltpu.sync_copy(rows_v, acc_sh.at[dst_v], add=True)

        plsc.subcore_barrier()
        pltpu.sync_copy(
            acc_sh.at[pl.ds(sid * RPW, RPW)],
            out_hbm.at[cid, pl.ds(sid * RPW, RPW)],
        )

    return k


def _s_from_deg(deg0_ref, deg1_ref):
    deg = deg0_ref[:, :1] + deg1_ref[:, :1] + 1.0
    return lax.rsqrt(deg)


def _mm_scale_body(x_ref, w_ref, deg0_ref, deg1_ref, out_ref):
    s = _s_from_deg(deg0_ref, deg1_ref)
    h = jnp.dot(
        x_ref[...], w_ref[...],
        preferred_element_type=jnp.float32, precision=lax.Precision.HIGHEST,
    )
    out_ref[...] = h * s


def _combine_mm_body(p0_ref, p1_ref, h2_ref, deg0_ref, deg1_ref, b_ref, w_ref,
                     out_ref):
    s = _s_from_deg(deg0_ref, deg1_ref)
    t = s * (p0_ref[...] + p1_ref[...] + h2_ref[...]) + b_ref[...]
    g = jnp.maximum(t, 0.0)
    h = jnp.dot(
        g, w_ref[...],
        preferred_element_type=jnp.float32, precision=lax.Precision.HIGHEST,
    )
    out_ref[...] = h * s


def _final_body(p0_ref, p1_ref, h2_ref, deg0_ref, deg1_ref, b_ref, out_ref):
    s = _s_from_deg(deg0_ref, deg1_ref)
    out_ref[...] = s * (p0_ref[...] + p1_ref[...] + h2_ref[...]) + b_ref[...]


def _row_spec(w):
    return pl.BlockSpec((ROW_TILE, w), lambda i: (i, 0))


def _full_spec(r, w):
    return pl.BlockSpec((r, w), lambda i: (0, 0))


_GRID = (N // ROW_TILE,)
_F32 = jnp.float32


@jax.jit
def kernel(x, edge_index, W1, b1, W2, b2):
    ei = edge_index.astype(jnp.int32)
    src, dst = ei[0], ei[1]
    e = src.shape[0]
    epw = -(-e // (NW * CHUNK)) * CHUNK
    e_pad = epw * NW
    if e_pad != e:
        pad = e_pad - e
        src = jnp.concatenate([src, jnp.zeros((pad,), jnp.int32)])
        dst = jnp.concatenate([dst, jnp.full((pad,), N, jnp.int32)])

    zeros_deg = jnp.zeros((N_PAD, DEG_W), _F32)
    zeros_acc = jnp.zeros((N_PAD, C), _F32)
    b1r = b1.reshape(1, C)
    b2r = b2.reshape(1, C)

    degp = _deg_kernel(epw)(dst, zeros_deg)
    deg0, deg1 = degp[0], degp[1]

    h2_1 = pl.pallas_call(
        _mm_scale_body,
        grid=_GRID,
        in_specs=[
            _row_spec(C), _full_spec(C, C), _row_spec(DEG_W), _row_spec(DEG_W)
        ],
        out_specs=_row_spec(C),
        out_shape=jax.ShapeDtypeStruct((N, C), _F32),
    )(x, W1, deg0, deg1)

    p = _agg_kernel(epw)(src, dst, h2_1, zeros_acc)
    p0, p1 = p[0, :N], p[1, :N]

    h2_2 = pl.pallas_call(
        _combine_mm_body,
        grid=_GRID,
        in_specs=[
            _row_spec(C), _row_spec(C), _row_spec(C),
            _row_spec(DEG_W), _row_spec(DEG_W),
            _full_spec(1, C), _full_spec(C, C),
        ],
        out_specs=_row_spec(C),
        out_shape=jax.ShapeDtypeStruct((N, C), _F32),
    )(p0, p1, h2_1, deg0, deg1, b1r, W2)

    q = _agg_kernel(epw)(src, dst, h2_2, zeros_acc)
    q0, q1 = q[0, :N], q[1, :N]

    out = pl.pallas_call(
        _final_body,
        grid=_GRID,
        in_specs=[
            _row_spec(C), _row_spec(C), _row_spec(C),
            _row_spec(DEG_W), _row_spec(DEG_W),
            _full_spec(1, C),
        ],
        out_specs=_row_spec(C),
        out_shape=jax.ShapeDtypeStruct((N, C), _F32),
    )(q0, q1, h2_2, deg0, deg1, b2r)

    return out
